# baseline (speedup 1.0000x reference)
import functools

import jax
import jax.numpy as jnp
from jax.experimental import pallas as pl
from jax.experimental.pallas import tpu as pltpu

T = 8192
D = 768
F = 128
E = 8
TB = 512


def _moe_kernel(x_ref, wr_ref, br_ref, w1_ref, b1_ref, w2_ref, b2_ref,
                sel_ref, out_ref, imp_ref, loss_ref, *, num_tiles):
    i = pl.program_id(0)
    x = x_ref[...]

    logits = jnp.dot(x, wr_ref[...], preferred_element_type=jnp.float32)
    logits = logits + br_ref[...]
    m = jnp.max(logits, axis=-1, keepdims=True)
    ex = jnp.exp(logits - m)
    scores = ex / jnp.sum(ex, axis=-1, keepdims=True)

    iota = jax.lax.broadcasted_iota(jnp.int32, (TB, E), 1)
    v1 = jnp.max(scores, axis=-1, keepdims=True)
    idx1 = jnp.min(jnp.where(scores == v1, iota, E), axis=-1, keepdims=True)
    mask1 = iota == idx1
    s2 = jnp.where(mask1, -jnp.inf, scores)
    v2 = jnp.max(s2, axis=-1, keepdims=True)
    idx2 = jnp.min(jnp.where(s2 == v2, iota, E), axis=-1, keepdims=True)
    w = jnp.where(mask1 | (iota == idx2), scores, 0.0)

    @pl.when(i == 0)
    def _init():
        imp_ref[...] = jnp.zeros_like(imp_ref)

    imp_ref[...] += jnp.sum(w, axis=0).reshape(1, E)

    xb = x.astype(jnp.bfloat16)
    h = jnp.concatenate(
        [jnp.dot(xb, w1_ref[e_i], preferred_element_type=jnp.float32)
         for e_i in range(E)],
        axis=-1)
    h = jnp.maximum(h + b1_ref[...], 0.0)

    wexp = jnp.dot(w, sel_ref[...], preferred_element_type=jnp.float32)
    hw = (h * wexp).astype(jnp.bfloat16)
    out = jnp.dot(hw, w2_ref[...], preferred_element_type=jnp.float32)
    out_ref[...] = out + jnp.dot(w, b2_ref[...],
                                 preferred_element_type=jnp.float32)

    @pl.when(i == num_tiles - 1)
    def _loss():
        imp = imp_ref[0, :]
        mean = jnp.mean(imp)
        var = jnp.sum((imp - mean) ** 2) / (E - 1)
        loss_ref[...] = (var / (mean * mean + 1e-9)).reshape(1, 1)


def kernel(x, Wr, br, W1, b1, W2, b2):
    num_tiles = T // TB
    sel = jnp.repeat(jnp.eye(E, dtype=jnp.float32), F, axis=1)
    out, imp, loss = pl.pallas_call(
        functools.partial(_moe_kernel, num_tiles=num_tiles),
        grid=(num_tiles,),
        in_specs=[
            pl.BlockSpec((TB, D), lambda i: (i, 0)),
            pl.BlockSpec((D, E), lambda i: (0, 0)),
            pl.BlockSpec((1, E), lambda i: (0, 0)),
            pl.BlockSpec((E, D, F), lambda i: (0, 0, 0)),
            pl.BlockSpec((1, E * F), lambda i: (0, 0)),
            pl.BlockSpec((E * F, D), lambda i: (0, 0)),
            pl.BlockSpec((E, D), lambda i: (0, 0)),
            pl.BlockSpec((E, E * F), lambda i: (0, 0)),
        ],
        out_specs=[
            pl.BlockSpec((TB, D), lambda i: (i, 0)),
            pl.BlockSpec((1, E), lambda i: (0, 0)),
            pl.BlockSpec((1, 1), lambda i: (0, 0)),
        ],
        out_shape=[
            jax.ShapeDtypeStruct((T, D), jnp.float32),
            jax.ShapeDtypeStruct((1, E), jnp.float32),
            jax.ShapeDtypeStruct((1, 1), jnp.float32),
        ],
        compiler_params=pltpu.CompilerParams(
            dimension_semantics=("arbitrary",),
        ),
    )(x, Wr, br.reshape(1, E), W1.astype(jnp.bfloat16),
      b1.reshape(1, E * F), W2.reshape(E * F, D).astype(jnp.bfloat16),
      b2, sel)
    del imp
    return out, loss[0, 0]


# speedup vs baseline: 1.1246x; 1.1246x over previous
import functools

import jax
import jax.numpy as jnp
from jax.experimental import pallas as pl
from jax.experimental.pallas import tpu as pltpu

T = 8192
D = 768
F = 128
E = 8
TB = 1024


def _moe_kernel(x_ref, wr_ref, br_ref, w1_ref, b1_ref, w2_ref, b2_ref,
                sel_ref, out_ref, imp_ref, loss_ref, *, num_tiles):
    i = pl.program_id(0)
    x = x_ref[...]

    logits = jnp.dot(x, wr_ref[...], preferred_element_type=jnp.float32)
    logits = logits + br_ref[...]
    m = jnp.max(logits, axis=-1, keepdims=True)
    ex = jnp.exp(logits - m)
    scores = ex / jnp.sum(ex, axis=-1, keepdims=True)

    iota = jax.lax.broadcasted_iota(jnp.int32, (TB, E), 1)
    v1 = jnp.max(scores, axis=-1, keepdims=True)
    idx1 = jnp.min(jnp.where(scores == v1, iota, E), axis=-1, keepdims=True)
    mask1 = iota == idx1
    s2 = jnp.where(mask1, -jnp.inf, scores)
    v2 = jnp.max(s2, axis=-1, keepdims=True)
    idx2 = jnp.min(jnp.where(s2 == v2, iota, E), axis=-1, keepdims=True)
    w = jnp.where(mask1 | (iota == idx2), scores, 0.0)

    @pl.when(i == 0)
    def _init():
        imp_ref[...] = jnp.zeros_like(imp_ref)

    imp_ref[...] += jnp.sum(w, axis=0).reshape(1, E)

    xb = x.astype(jnp.bfloat16)
    h = jnp.concatenate(
        [jnp.dot(xb, w1_ref[e_i].astype(jnp.bfloat16),
                 preferred_element_type=jnp.float32) for e_i in range(E)],
        axis=-1)
    h = jnp.maximum(h + b1_ref[...], 0.0)

    wexp = jnp.dot(w, sel_ref[...], preferred_element_type=jnp.float32)
    hw = (h * wexp).astype(jnp.bfloat16)
    out = jnp.dot(hw, w2_ref[...].astype(jnp.bfloat16),
                  preferred_element_type=jnp.float32)
    out_ref[...] = out + jnp.dot(w, b2_ref[...],
                                 preferred_element_type=jnp.float32)

    @pl.when(i == num_tiles - 1)
    def _loss():
        imp = imp_ref[0, :]
        mean = jnp.mean(imp)
        var = jnp.sum((imp - mean) ** 2) / (E - 1)
        loss_ref[...] = (var / (mean * mean + 1e-9)).reshape(1, 1)


def kernel(x, Wr, br, W1, b1, W2, b2):
    num_tiles = T // TB
    sel = jnp.repeat(jnp.eye(E, dtype=jnp.float32), F, axis=1)
    out, imp, loss = pl.pallas_call(
        functools.partial(_moe_kernel, num_tiles=num_tiles),
        grid=(num_tiles,),
        in_specs=[
            pl.BlockSpec((TB, D), lambda i: (i, 0)),
            pl.BlockSpec((D, E), lambda i: (0, 0)),
            pl.BlockSpec((1, E), lambda i: (0, 0)),
            pl.BlockSpec((E, D, F), lambda i: (0, 0, 0)),
            pl.BlockSpec((1, E * F), lambda i: (0, 0)),
            pl.BlockSpec((E * F, D), lambda i: (0, 0)),
            pl.BlockSpec((E, D), lambda i: (0, 0)),
            pl.BlockSpec((E, E * F), lambda i: (0, 0)),
        ],
        out_specs=[
            pl.BlockSpec((TB, D), lambda i: (i, 0)),
            pl.BlockSpec((1, E), lambda i: (0, 0)),
            pl.BlockSpec((1, 1), lambda i: (0, 0)),
        ],
        out_shape=[
            jax.ShapeDtypeStruct((T, D), jnp.float32),
            jax.ShapeDtypeStruct((1, E), jnp.float32),
            jax.ShapeDtypeStruct((1, 1), jnp.float32),
        ],
        compiler_params=pltpu.CompilerParams(
            dimension_semantics=("arbitrary",),
        ),
    )(x, Wr, br.reshape(1, E), W1, b1.reshape(1, E * F),
      W2.reshape(E * F, D), b2, sel)
    del imp
    return out, loss[0, 0]


# speedup vs baseline: 1.1306x; 1.0053x over previous
import functools

import jax
import jax.numpy as jnp
from jax.experimental import pallas as pl
from jax.experimental.pallas import tpu as pltpu

T = 8192
D = 768
F = 128
E = 8
TB = 2048


def _moe_kernel(x_ref, wr_ref, br_ref, w1_ref, b1_ref, w2_ref, b2_ref,
                sel_ref, out_ref, imp_ref, loss_ref, *, num_tiles):
    i = pl.program_id(0)
    x = x_ref[...]

    logits = jnp.dot(x, wr_ref[...], preferred_element_type=jnp.float32)
    logits = logits + br_ref[...]
    m = jnp.max(logits, axis=-1, keepdims=True)
    ex = jnp.exp(logits - m)
    scores = ex / jnp.sum(ex, axis=-1, keepdims=True)

    iota = jax.lax.broadcasted_iota(jnp.int32, (TB, E), 1)
    v1 = jnp.max(scores, axis=-1, keepdims=True)
    idx1 = jnp.min(jnp.where(scores == v1, iota, E), axis=-1, keepdims=True)
    mask1 = iota == idx1
    s2 = jnp.where(mask1, -jnp.inf, scores)
    v2 = jnp.max(s2, axis=-1, keepdims=True)
    idx2 = jnp.min(jnp.where(s2 == v2, iota, E), axis=-1, keepdims=True)
    w = jnp.where(mask1 | (iota == idx2), scores, 0.0)

    @pl.when(i == 0)
    def _init():
        imp_ref[...] = jnp.zeros_like(imp_ref)

    imp_ref[...] += jnp.sum(w, axis=0).reshape(1, E)

    xb = x.astype(jnp.bfloat16)
    h = jnp.concatenate(
        [jnp.dot(xb, w1_ref[e_i].astype(jnp.bfloat16),
                 preferred_element_type=jnp.float32) for e_i in range(E)],
        axis=-1)
    h = jnp.maximum(h + b1_ref[...], 0.0)

    wexp = jnp.dot(w, sel_ref[...], preferred_element_type=jnp.float32)
    hw = (h * wexp).astype(jnp.bfloat16)
    out = jnp.dot(hw, w2_ref[...].astype(jnp.bfloat16),
                  preferred_element_type=jnp.float32)
    out_ref[...] = out + jnp.dot(w, b2_ref[...],
                                 preferred_element_type=jnp.float32)

    @pl.when(i == num_tiles - 1)
    def _loss():
        imp = imp_ref[0, :]
        mean = jnp.mean(imp)
        var = jnp.sum((imp - mean) ** 2) / (E - 1)
        loss_ref[...] = (var / (mean * mean + 1e-9)).reshape(1, 1)


def kernel(x, Wr, br, W1, b1, W2, b2):
    num_tiles = T // TB
    sel = jnp.repeat(jnp.eye(E, dtype=jnp.float32), F, axis=1)
    out, imp, loss = pl.pallas_call(
        functools.partial(_moe_kernel, num_tiles=num_tiles),
        grid=(num_tiles,),
        in_specs=[
            pl.BlockSpec((TB, D), lambda i: (i, 0)),
            pl.BlockSpec((D, E), lambda i: (0, 0)),
            pl.BlockSpec((1, E), lambda i: (0, 0)),
            pl.BlockSpec((E, D, F), lambda i: (0, 0, 0)),
            pl.BlockSpec((1, E * F), lambda i: (0, 0)),
            pl.BlockSpec((E * F, D), lambda i: (0, 0)),
            pl.BlockSpec((E, D), lambda i: (0, 0)),
            pl.BlockSpec((E, E * F), lambda i: (0, 0)),
        ],
        out_specs=[
            pl.BlockSpec((TB, D), lambda i: (i, 0)),
            pl.BlockSpec((1, E), lambda i: (0, 0)),
            pl.BlockSpec((1, 1), lambda i: (0, 0)),
        ],
        out_shape=[
            jax.ShapeDtypeStruct((T, D), jnp.float32),
            jax.ShapeDtypeStruct((1, E), jnp.float32),
            jax.ShapeDtypeStruct((1, 1), jnp.float32),
        ],
        compiler_params=pltpu.CompilerParams(
            dimension_semantics=("arbitrary",),
        ),
    )(x, Wr, br.reshape(1, E), W1, b1.reshape(1, E * F),
      W2.reshape(E * F, D), b2, sel)
    del imp
    return out, loss[0, 0]


# speedup vs baseline: 1.5752x; 1.3932x over previous
import functools

import jax
import jax.numpy as jnp
from jax.experimental import pallas as pl
from jax.experimental.pallas import tpu as pltpu

T = 8192
D = 768
F = 128
E = 8
TB = 2048
EF = E * F
PW = 128


def _moe_kernel(x_ref, wr_ref, br_ref, w1_ref, b1_ref, w2_ref, b2_ref,
                sel_ref, out_ref, imp_ref, loss_ref, w1c_ref, w2a_ref,
                *, num_tiles):
    i = pl.program_id(0)

    @pl.when(i == 0)
    def _stage():
        for e_i in range(E):
            w1c_ref[:, e_i * F:(e_i + 1) * F] = (
                w1_ref[e_i].astype(jnp.bfloat16))
        w2a_ref[0:EF, :] = w2_ref[...].astype(jnp.bfloat16)
        w2a_ref[EF:EF + E, :] = b2_ref[...].astype(jnp.bfloat16)
        w2a_ref[EF + E:, :] = jnp.zeros((PW - E, D), jnp.bfloat16)
        imp_ref[...] = jnp.zeros_like(imp_ref)

    x = x_ref[...]

    logits = jnp.dot(x, wr_ref[...], preferred_element_type=jnp.float32)
    logits = logits + br_ref[...]
    m = jnp.max(logits, axis=-1, keepdims=True)
    ex = jnp.exp(logits - m)
    scores = ex / jnp.sum(ex, axis=-1, keepdims=True)

    iota = jax.lax.broadcasted_iota(jnp.int32, (TB, E), 1)
    v1 = jnp.max(scores, axis=-1, keepdims=True)
    idx1 = jnp.min(jnp.where(scores == v1, iota, E), axis=-1, keepdims=True)
    mask1 = iota == idx1
    s2 = jnp.where(mask1, -jnp.inf, scores)
    v2 = jnp.max(s2, axis=-1, keepdims=True)
    idx2 = jnp.min(jnp.where(s2 == v2, iota, E), axis=-1, keepdims=True)
    w = jnp.where(mask1 | (iota == idx2), scores, 0.0)

    imp_ref[...] += jnp.sum(w, axis=0).reshape(1, E)

    xb = x.astype(jnp.bfloat16)
    h = jnp.dot(xb, w1c_ref[...], preferred_element_type=jnp.float32)
    h = jnp.maximum(h + b1_ref[...], 0.0)

    wexp = jnp.dot(w, sel_ref[...], preferred_element_type=jnp.float32)
    hw = (h * wexp).astype(jnp.bfloat16)
    wpad = jnp.pad(w, ((0, 0), (0, PW - E))).astype(jnp.bfloat16)
    hcat = jnp.concatenate([hw, wpad], axis=-1)
    out_ref[...] = jnp.dot(hcat, w2a_ref[...],
                           preferred_element_type=jnp.float32)

    @pl.when(i == num_tiles - 1)
    def _loss():
        imp = imp_ref[0, :]
        mean = jnp.mean(imp)
        var = jnp.sum((imp - mean) ** 2) / (E - 1)
        loss_ref[...] = (var / (mean * mean + 1e-9)).reshape(1, 1)


def kernel(x, Wr, br, W1, b1, W2, b2):
    num_tiles = T // TB
    sel = jnp.repeat(jnp.eye(E, dtype=jnp.float32), F, axis=1)
    out, imp, loss = pl.pallas_call(
        functools.partial(_moe_kernel, num_tiles=num_tiles),
        grid=(num_tiles,),
        in_specs=[
            pl.BlockSpec((TB, D), lambda i: (i, 0)),
            pl.BlockSpec((D, E), lambda i: (0, 0)),
            pl.BlockSpec((1, E), lambda i: (0, 0)),
            pl.BlockSpec((E, D, F), lambda i: (0, 0, 0)),
            pl.BlockSpec((1, EF), lambda i: (0, 0)),
            pl.BlockSpec((EF, D), lambda i: (0, 0)),
            pl.BlockSpec((E, D), lambda i: (0, 0)),
            pl.BlockSpec((E, EF), lambda i: (0, 0)),
        ],
        out_specs=[
            pl.BlockSpec((TB, D), lambda i: (i, 0)),
            pl.BlockSpec((1, E), lambda i: (0, 0)),
            pl.BlockSpec((1, 1), lambda i: (0, 0)),
        ],
        out_shape=[
            jax.ShapeDtypeStruct((T, D), jnp.float32),
            jax.ShapeDtypeStruct((1, E), jnp.float32),
            jax.ShapeDtypeStruct((1, 1), jnp.float32),
        ],
        scratch_shapes=[
            pltpu.VMEM((D, EF), jnp.bfloat16),
            pltpu.VMEM((EF + PW, D), jnp.bfloat16),
        ],
        compiler_params=pltpu.CompilerParams(
            dimension_semantics=("arbitrary",),
        ),
    )(x, Wr, br.reshape(1, E), W1, b1.reshape(1, EF),
      W2.reshape(EF, D), b2, sel)
    del imp
    return out, loss[0, 0]


# speedup vs baseline: 1.6342x; 1.0374x over previous
import functools

import jax
import jax.numpy as jnp
from jax.experimental import pallas as pl
from jax.experimental.pallas import tpu as pltpu

T = 8192
D = 768
F = 128
E = 8
TB = 2048
EF = E * F
PW = 128


def _moe_kernel(x_ref, wr_ref, br_ref, w1_ref, b1_ref, w2_ref, b2_ref,
                sel_ref, out_ref, imp_ref, loss_ref, w1c_ref, w2a_ref,
                *, num_tiles):
    i = pl.program_id(0)

    @pl.when(i == 0)
    def _stage():
        for e_i in range(E):
            w1c_ref[:, e_i * F:(e_i + 1) * F] = (
                w1_ref[e_i].astype(jnp.bfloat16))
        w2a_ref[0:EF, :] = w2_ref[...].astype(jnp.bfloat16)
        w2a_ref[EF:EF + E, :] = b2_ref[...].astype(jnp.bfloat16)
        w2a_ref[EF + E:, :] = jnp.zeros((PW - E, D), jnp.bfloat16)
        imp_ref[...] = jnp.zeros_like(imp_ref)

    x = x_ref[...]

    logits = jnp.dot(x, wr_ref[...], preferred_element_type=jnp.float32)
    logits = logits + br_ref[...]
    lt = logits.T
    m = jnp.max(lt, axis=0, keepdims=True)
    ex = jnp.exp(lt - m)
    scores = ex / jnp.sum(ex, axis=0, keepdims=True)

    iota = jax.lax.broadcasted_iota(jnp.int32, (E, TB), 0)
    v1 = jnp.max(scores, axis=0, keepdims=True)
    idx1 = jnp.min(jnp.where(scores == v1, iota, E), axis=0, keepdims=True)
    mask1 = iota == idx1
    s2 = jnp.where(mask1, -jnp.inf, scores)
    v2 = jnp.max(s2, axis=0, keepdims=True)
    idx2 = jnp.min(jnp.where(s2 == v2, iota, E), axis=0, keepdims=True)
    wt = jnp.where(mask1 | (iota == idx2), scores, 0.0)

    imp_ref[...] += jnp.sum(wt, axis=1, keepdims=True)
    w = wt.T

    xb = x.astype(jnp.bfloat16)
    h = jnp.dot(xb, w1c_ref[...], preferred_element_type=jnp.float32)
    h = jnp.maximum(h + b1_ref[...], 0.0)

    wexp = jnp.dot(w, sel_ref[...], preferred_element_type=jnp.float32)
    hw = (h * wexp).astype(jnp.bfloat16)
    wpad = jnp.pad(w, ((0, 0), (0, PW - E))).astype(jnp.bfloat16)
    hcat = jnp.concatenate([hw, wpad], axis=-1)
    out_ref[...] = jnp.dot(hcat, w2a_ref[...],
                           preferred_element_type=jnp.float32)

    @pl.when(i == num_tiles - 1)
    def _loss():
        imp = imp_ref[...]
        mean = jnp.sum(imp) / E
        var = jnp.sum((imp - mean) ** 2) / (E - 1)
        loss_ref[...] = (var / (mean * mean + 1e-9)).reshape(1, 1)


def kernel(x, Wr, br, W1, b1, W2, b2):
    num_tiles = T // TB
    sel = jnp.repeat(jnp.eye(E, dtype=jnp.float32), F, axis=1)
    out, imp, loss = pl.pallas_call(
        functools.partial(_moe_kernel, num_tiles=num_tiles),
        grid=(num_tiles,),
        in_specs=[
            pl.BlockSpec((TB, D), lambda i: (i, 0)),
            pl.BlockSpec((D, E), lambda i: (0, 0)),
            pl.BlockSpec((1, E), lambda i: (0, 0)),
            pl.BlockSpec((E, D, F), lambda i: (0, 0, 0)),
            pl.BlockSpec((1, EF), lambda i: (0, 0)),
            pl.BlockSpec((EF, D), lambda i: (0, 0)),
            pl.BlockSpec((E, D), lambda i: (0, 0)),
            pl.BlockSpec((E, EF), lambda i: (0, 0)),
        ],
        out_specs=[
            pl.BlockSpec((TB, D), lambda i: (i, 0)),
            pl.BlockSpec((E, 1), lambda i: (0, 0)),
            pl.BlockSpec((1, 1), lambda i: (0, 0)),
        ],
        out_shape=[
            jax.ShapeDtypeStruct((T, D), jnp.float32),
            jax.ShapeDtypeStruct((E, 1), jnp.float32),
            jax.ShapeDtypeStruct((1, 1), jnp.float32),
        ],
        scratch_shapes=[
            pltpu.VMEM((D, EF), jnp.bfloat16),
            pltpu.VMEM((EF + PW, D), jnp.bfloat16),
        ],
        compiler_params=pltpu.CompilerParams(
            dimension_semantics=("arbitrary",),
        ),
    )(x, Wr, br.reshape(1, E), W1, b1.reshape(1, EF),
      W2.reshape(EF, D), b2, sel)
    del imp
    return out, loss[0, 0]


# speedup vs baseline: 1.6505x; 1.0100x over previous
import functools

import jax
import jax.numpy as jnp
from jax.experimental import pallas as pl
from jax.experimental.pallas import tpu as pltpu

T = 8192
D = 768
F = 128
E = 8
TB = 1024
EF = E * F
PW = 128


def _moe_kernel(x_ref, wr_ref, br_ref, w1_ref, b1_ref, w2_ref, b2_ref,
                sel_ref, out_ref, imp_ref, loss_ref, w1c_ref, w2a_ref,
                *, num_tiles):
    i = pl.program_id(0)

    @pl.when(i == 0)
    def _stage():
        for e_i in range(E):
            w1c_ref[:, e_i * F:(e_i + 1) * F] = (
                w1_ref[e_i].astype(jnp.bfloat16))
        w2a_ref[0:EF, :] = w2_ref[...].astype(jnp.bfloat16)
        w2a_ref[EF:EF + E, :] = b2_ref[...].astype(jnp.bfloat16)
        w2a_ref[EF + E:, :] = jnp.zeros((PW - E, D), jnp.bfloat16)
        imp_ref[...] = jnp.zeros_like(imp_ref)

    x = x_ref[...]

    logits = jnp.dot(x, wr_ref[...], preferred_element_type=jnp.float32)
    logits = logits + br_ref[...]
    lt = logits.T
    m = jnp.max(lt, axis=0, keepdims=True)
    ex = jnp.exp(lt - m)
    scores = ex / jnp.sum(ex, axis=0, keepdims=True)

    iota = jax.lax.broadcasted_iota(jnp.int32, (E, TB), 0)
    v1 = jnp.max(scores, axis=0, keepdims=True)
    idx1 = jnp.min(jnp.where(scores == v1, iota, E), axis=0, keepdims=True)
    mask1 = iota == idx1
    s2 = jnp.where(mask1, -jnp.inf, scores)
    v2 = jnp.max(s2, axis=0, keepdims=True)
    idx2 = jnp.min(jnp.where(s2 == v2, iota, E), axis=0, keepdims=True)
    wt = jnp.where(mask1 | (iota == idx2), scores, 0.0)

    imp_ref[...] += jnp.sum(wt, axis=1, keepdims=True)
    w = wt.T

    xb = x.astype(jnp.bfloat16)
    h = jnp.dot(xb, w1c_ref[...], preferred_element_type=jnp.float32)
    h = jnp.maximum(h + b1_ref[...], 0.0)

    wexp = jnp.dot(w, sel_ref[...], preferred_element_type=jnp.float32)
    hw = (h * wexp).astype(jnp.bfloat16)
    wpad = jnp.pad(w, ((0, 0), (0, PW - E))).astype(jnp.bfloat16)
    hcat = jnp.concatenate([hw, wpad], axis=-1)
    out_ref[...] = jnp.dot(hcat, w2a_ref[...],
                           preferred_element_type=jnp.float32)

    @pl.when(i == num_tiles - 1)
    def _loss():
        imp = imp_ref[...]
        mean = jnp.sum(imp) / E
        var = jnp.sum((imp - mean) ** 2) / (E - 1)
        loss_ref[...] = (var / (mean * mean + 1e-9)).reshape(1, 1)


def kernel(x, Wr, br, W1, b1, W2, b2):
    num_tiles = T // TB
    sel = jnp.repeat(jnp.eye(E, dtype=jnp.float32), F, axis=1)
    out, imp, loss = pl.pallas_call(
        functools.partial(_moe_kernel, num_tiles=num_tiles),
        grid=(num_tiles,),
        in_specs=[
            pl.BlockSpec((TB, D), lambda i: (i, 0)),
            pl.BlockSpec((D, E), lambda i: (0, 0)),
            pl.BlockSpec((1, E), lambda i: (0, 0)),
            pl.BlockSpec((E, D, F), lambda i: (0, 0, 0)),
            pl.BlockSpec((1, EF), lambda i: (0, 0)),
            pl.BlockSpec((EF, D), lambda i: (0, 0)),
            pl.BlockSpec((E, D), lambda i: (0, 0)),
            pl.BlockSpec((E, EF), lambda i: (0, 0)),
        ],
        out_specs=[
            pl.BlockSpec((TB, D), lambda i: (i, 0)),
            pl.BlockSpec((E, 1), lambda i: (0, 0)),
            pl.BlockSpec((1, 1), lambda i: (0, 0)),
        ],
        out_shape=[
            jax.ShapeDtypeStruct((T, D), jnp.float32),
            jax.ShapeDtypeStruct((E, 1), jnp.float32),
            jax.ShapeDtypeStruct((1, 1), jnp.float32),
        ],
        scratch_shapes=[
            pltpu.VMEM((D, EF), jnp.bfloat16),
            pltpu.VMEM((EF + PW, D), jnp.bfloat16),
        ],
        compiler_params=pltpu.CompilerParams(
            dimension_semantics=("arbitrary",),
        ),
    )(x, Wr, br.reshape(1, E), W1, b1.reshape(1, EF),
      W2.reshape(EF, D), b2, sel)
    del imp
    return out, loss[0, 0]
